# baseline (speedup 1.0000x reference)
import functools

import jax
import jax.numpy as jnp
from jax import lax
from jax.experimental import pallas as pl
from jax.experimental.pallas import tpu as pltpu
from jax.experimental.pallas import tpu_sc as plsc

_CR = 16
_XB = 3
_PB = 2


def _sc_kernel(B, S, D):
    info = plsc.get_sparse_core_info()
    NC, NS = info.num_cores, info.num_subcores
    NW = NC * NS
    RW = S // NW
    NCH = RW // _CR
    T = NCH * B

    mesh = plsc.VectorSubcoreMesh(core_axis_name="c", subcore_axis_name="s")

    @functools.partial(
        pl.kernel,
        out_type=jax.ShapeDtypeStruct((B * S, D), jnp.float32),
        mesh=mesh,
        scratch_types=[
            [pltpu.VMEM((_CR, D), jnp.float32)] * _XB,
            [pltpu.VMEM((_CR, D), jnp.float32)] * _PB,
            pltpu.VMEM_SHARED((NS * 2 * _CR, D), jnp.float32),
            [pltpu.SemaphoreType.DMA] * _XB,
            [pltpu.SemaphoreType.DMA] * _PB,
            [pltpu.SemaphoreType.DMA] * 2,
            [pltpu.SemaphoreType.DMA] * 2,
        ],
    )
    def body(xf, pf, of, xbufs, pbufs, shared, xsem, psem, bsem, osem):
        wid = lax.axis_index("s") * NC + lax.axis_index("c")
        sid = lax.axis_index("s")
        base = wid * RW

        def shslot(k):
            return shared.at[pl.ds((sid * 2 + k) * _CR, _CR)]

        def xrow(t):
            c, b = divmod(t, B)
            return b * S + base + c * _CR

        xd = [None] * T
        xb = [None] * T
        oh = [None] * T
        oh_waited = [False] * T
        pd = [None] * NCH

        def start_x(t):
            s = t % _XB
            xd[t] = pltpu.async_copy(
                xf.at[pl.ds(xrow(t), _CR)], xbufs[s], xsem[s])

        def start_p(c):
            s = c % _PB
            pd[c] = pltpu.async_copy(
                pf.at[pl.ds(base + c * _CR, _CR)], pbufs[s], psem[s])

        start_p(0)
        for t in range(min(_XB - 1, T)):
            start_x(t)

        for t in range(T):
            c, b = divmod(t, B)
            if b == 0 and c + 1 < NCH:
                start_p(c + 1)
            xd[t].wait()
            if b == 0:
                pd[c].wait()
            xv = xbufs[t % _XB]
            pv = pbufs[c % _PB]

            @plsc.parallel_loop(0, _CR * D, 16, unroll=16)
            def _(i):
                r = i // D
                col = i % D
                plsc.addupdate(xv.at[r, pl.ds(col, 16)],
                               pv[r, pl.ds(col, 16)])

            k = t % 2
            if t - 2 >= 0:
                oh[t - 2].wait()
                oh_waited[t - 2] = True
            xb[t] = pltpu.async_copy(xv, shslot(k), bsem[k])
            if t - 1 >= 0:
                xb[t - 1].wait()
                kk = (t - 1) % 2
                oh[t - 1] = pltpu.async_copy(
                    shslot(kk), of.at[pl.ds(xrow(t - 1), _CR)], osem[kk])
            if t + 2 < T:
                start_x(t + 2)

        xb[T - 1].wait()
        kk = (T - 1) % 2
        oh[T - 1] = pltpu.async_copy(
            shslot(kk), of.at[pl.ds(xrow(T - 1), _CR)], osem[kk])
        for t in range(T):
            if oh[t] is not None and not oh_waited[t]:
                oh[t].wait()

    return body


def _tc_body(x_ref, p_ref, o_ref):
    o_ref[...] = x_ref[...] + p_ref[...][None, :, :]


def _tc_kernel(x, pos):
    B, S, D = x.shape
    ts = 256 if S % 256 == 0 else S
    return pl.pallas_call(
        _tc_body,
        grid=(S // ts,),
        in_specs=[
            pl.BlockSpec((B, ts, D), lambda i: (0, i, 0)),
            pl.BlockSpec((ts, D), lambda i: (i, 0)),
        ],
        out_specs=pl.BlockSpec((B, ts, D), lambda i: (0, i, 0)),
        out_shape=jax.ShapeDtypeStruct((B, S, D), x.dtype),
    )(x, pos)


def kernel(x, pos_embedding):
    B, S, D = x.shape
    pos = pos_embedding[:S]
    info = plsc.get_sparse_core_info()
    NW = info.num_cores * info.num_subcores
    rows_per_worker = S // NW
    if (x.dtype == jnp.float32 and S % NW == 0
            and rows_per_worker % _CR == 0 and D % 16 == 0):
        out = _sc_kernel(B, S, D)(x.reshape(B * S, D), pos)
        return out.reshape(B, S, D)
    return _tc_kernel(x, pos)


# speedup vs baseline: 1.0057x; 1.0057x over previous
import functools

import jax
import jax.numpy as jnp
from jax import lax
from jax.experimental import pallas as pl
from jax.experimental.pallas import tpu as pltpu
from jax.experimental.pallas import tpu_sc as plsc

_CR = 16
_XB = 4
_PB = 2
_OB = 2


def _sc_kernel(B, S, D):
    info = plsc.get_sparse_core_info()
    NC, NS = info.num_cores, info.num_subcores
    NW = NC * NS
    RW = S // NW
    NCH = RW // _CR
    T = NCH * B

    mesh = plsc.VectorSubcoreMesh(core_axis_name="c", subcore_axis_name="s")

    @functools.partial(
        pl.kernel,
        out_type=jax.ShapeDtypeStruct((B * S, D), jnp.float32),
        mesh=mesh,
        scratch_types=[
            [pltpu.VMEM((_CR, D), jnp.float32)] * _XB,
            [pltpu.VMEM((_CR, D), jnp.float32)] * _PB,
            pltpu.VMEM_SHARED((NS * _OB * _CR, D), jnp.float32),
            [pltpu.SemaphoreType.DMA] * _XB,
            [pltpu.SemaphoreType.DMA] * _PB,
            [pltpu.SemaphoreType.DMA] * _OB,
            [pltpu.SemaphoreType.DMA] * _OB,
        ],
    )
    def body(xf, pf, of, xbufs, pbufs, shared, xsem, psem, bsem, osem):
        wid = lax.axis_index("s") * NC + lax.axis_index("c")
        sid = lax.axis_index("s")
        base = wid * RW

        def shslot(k):
            return shared.at[pl.ds((sid * _OB + k) * _CR, _CR)]

        def xrow(t):
            c, b = divmod(t, B)
            return b * S + base + c * _CR

        xd = [None] * T
        xb = [None] * T
        oh = [None] * T
        oh_waited = [False] * T
        pd = [None] * NCH

        def start_x(t):
            s = t % _XB
            xd[t] = pltpu.async_copy(
                xf.at[pl.ds(xrow(t), _CR)], xbufs[s], xsem[s])

        def start_p(c):
            s = c % _PB
            pd[c] = pltpu.async_copy(
                pf.at[pl.ds(base + c * _CR, _CR)], pbufs[s], psem[s])

        start_p(0)
        for t in range(min(_XB - 1, T)):
            start_x(t)

        for t in range(T):
            c, b = divmod(t, B)
            if b == 0 and c + 1 < NCH:
                start_p(c + 1)
            xd[t].wait()
            if b == 0:
                pd[c].wait()
            xv = xbufs[t % _XB]
            pv = pbufs[c % _PB]

            @plsc.parallel_loop(0, _CR * D, 16, unroll=16)
            def _(i):
                r = i // D
                col = i % D
                plsc.addupdate(xv.at[r, pl.ds(col, 16)],
                               pv[r, pl.ds(col, 16)])

            k = t % _OB
            if t - _OB >= 0:
                oh[t - _OB].wait()
                oh_waited[t - _OB] = True
            xb[t] = pltpu.async_copy(xv, shslot(k), bsem[k])
            if t - 1 >= 0:
                xb[t - 1].wait()
                kk = (t - 1) % _OB
                oh[t - 1] = pltpu.async_copy(
                    shslot(kk), of.at[pl.ds(xrow(t - 1), _CR)], osem[kk])
            if t + _XB - 1 < T:
                start_x(t + _XB - 1)

        xb[T - 1].wait()
        kk = (T - 1) % _OB
        oh[T - 1] = pltpu.async_copy(
            shslot(kk), of.at[pl.ds(xrow(T - 1), _CR)], osem[kk])
        for t in range(T):
            if oh[t] is not None and not oh_waited[t]:
                oh[t].wait()

    return body


def _tc_body(x_ref, p_ref, o_ref):
    o_ref[...] = x_ref[...] + p_ref[...][None, :, :]


def _tc_kernel(x, pos):
    B, S, D = x.shape
    ts = 256 if S % 256 == 0 else S
    return pl.pallas_call(
        _tc_body,
        grid=(S // ts,),
        in_specs=[
            pl.BlockSpec((B, ts, D), lambda i: (0, i, 0)),
            pl.BlockSpec((ts, D), lambda i: (i, 0)),
        ],
        out_specs=pl.BlockSpec((B, ts, D), lambda i: (0, i, 0)),
        out_shape=jax.ShapeDtypeStruct((B, S, D), x.dtype),
    )(x, pos)


def kernel(x, pos_embedding):
    B, S, D = x.shape
    pos = pos_embedding[:S]
    info = plsc.get_sparse_core_info()
    NW = info.num_cores * info.num_subcores
    rows_per_worker = S // NW
    if (x.dtype == jnp.float32 and S % NW == 0
            and rows_per_worker % _CR == 0 and D % 16 == 0):
        out = _sc_kernel(B, S, D)(x.reshape(B * S, D), pos)
        return out.reshape(B, S, D)
    return _tc_kernel(x, pos)
